# baseline (speedup 1.0000x reference)
import jax
import jax.numpy as jnp
from jax import lax
from jax.experimental import pallas as pl
from jax.experimental.pallas import tpu as pltpu
from jax.experimental.pallas import tpu_sc as plsc

EPS = 1e-5
P = 64
NS_OUT = 128
NC, NSUB = 2, 16
NW = NC * NSUB
CH = 128


def _sampler_call(Lf, Wf):
    R = Lf.shape[0] // P
    rpw = R // NW
    nch = rpw // CH

    mesh = plsc.VectorSubcoreMesh(
        core_axis_name="c", subcore_axis_name="s",
        num_cores=NC, num_subcores=NSUB)

    def body(len_hbm, w_hbm, out_hbm, LbA, WbA, ObA, LbB, WbB, ObB,
             cdfb, midsb, histb, isemA, isemB, osemA, osemB):
        wid = lax.axis_index("s") * NC + lax.axis_index("c")
        base = wid * rpw

        iota = lax.iota(jnp.int32, 16)
        iof = iota.astype(jnp.float32)
        u = [iof * (1.0 / 63.0) + (16.0 * c / 63.0) for c in range(4)]
        ones_i = jnp.ones((16,), jnp.int32)
        zeros_i = jnp.zeros((16,), jnp.int32)
        ones_f = jnp.ones((16,), jnp.float32)
        full62 = jnp.full((16,), 62, jnp.int32)
        full63 = jnp.full((16,), 63, jnp.int32)
        mask3 = iota < 15

        def make_ray_body(Lb, Wb, Ob):
          def ray_body(r):
            rb = r * P
            rbv = jnp.full((16,), rb, jnp.int32)
            l = [Lb[pl.ds(rb + 16 * c, 16)] for c in range(4)]
            lsh = [Lb[pl.ds(rb + 16 * c + 1, 16)] for c in range(3)]
            lsh.append(plsc.load_gather(
                Lb, [rbv + jnp.minimum(iota + 49, 63)]))
            for c in range(4):
                midsb[pl.ds(rb + 16 * c, 16)] = 0.5 * (l[c] + lsh[c])
            cs = []
            for c in range(4):
                w = Wb[pl.ds(rb + 16 * c, 16)] + EPS
                cc = plsc.cumsum(w)
                cdfb[pl.ds(rb + 16 * c, 16)] = cc
                cs.append(cc)
            s0 = plsc.load_gather(cdfb, [rbv])
            g15 = plsc.load_gather(cdfb, [rbv + 15])
            g31 = plsc.load_gather(cdfb, [rbv + 31])
            g47 = plsc.load_gather(cdfb, [rbv + 47])
            g62 = plsc.load_gather(cdfb, [rbv + 62])
            off = [None, g15, g15 + g31, g15 + g31 + g47]
            invv = ones_f / ((off[3] + g62) - s0)
            pos = []
            for c in range(4):
                if c == 0:
                    cdf_c = (cs[c] - s0) * invv
                else:
                    cdf_c = (cs[c] + (off[c] - s0)) * invv
                cdfb[pl.ds(rb + 16 * c, 16)] = cdf_c
                p = cdf_c * 63.0
                ti = p.astype(jnp.int32)
                tf = ti.astype(jnp.float32)
                po = ti + jnp.where(tf < p, ones_i, zeros_i)
                po = jnp.minimum(po, full63)
                pos.append(po)
            for c in range(4):
                histb[pl.ds(rb + 16 * c, 16)] = zeros_i
            for c in range(4):
                plsc.addupdate_scatter(
                    histb, [rbv + pos[c]], ones_i,
                    mask=None if c < 3 else mask3)
            ics = []
            for c in range(4):
                ic = plsc.cumsum(histb[pl.ds(rb + 16 * c, 16)])
                histb[pl.ds(rb + 16 * c, 16)] = ic
                ics.append(ic)
            h15 = plsc.load_gather(histb, [rbv + 15])
            h31 = plsc.load_gather(histb, [rbv + 31])
            h47 = plsc.load_gather(histb, [rbv + 47])
            ioff = [None, h15, h15 + h31, h15 + h31 + h47]
            samples = []
            for c in range(4):
                ind = ics[c] if c == 0 else ics[c] + ioff[c]
                below = rbv + (ind - ones_i)
                above = rbv + jnp.minimum(ind, full62)
                g0 = plsc.load_gather(cdfb, [below])
                g1 = plsc.load_gather(cdfb, [above])
                m0 = plsc.load_gather(midsb, [below])
                m1 = plsc.load_gather(midsb, [above])
                den = g1 - g0
                den = jnp.where(den < EPS, ones_f, den)
                t = (u[c] - g0) / den
                samples.append(m0 + t * (m1 - m0))
            v = l + [lax.rev(samples[3], (0,)), lax.rev(samples[2], (0,)),
                     lax.rev(samples[1], (0,)), lax.rev(samples[0], (0,))]
            for i in range(4):
                a, b = v[i], v[i + 4]
                v[i], v[i + 4] = jnp.minimum(a, b), jnp.maximum(a, b)
            for h in (0, 4):
                for i in (0, 1):
                    a, b = v[h + i], v[h + i + 2]
                    v[h + i], v[h + i + 2] = jnp.minimum(a, b), jnp.maximum(a, b)
            for i in (0, 2, 4, 6):
                a, b = v[i], v[i + 1]
                v[i], v[i + 1] = jnp.minimum(a, b), jnp.maximum(a, b)
            for i in range(8):
                Ob[r, pl.ds(16 * i, 16)] = jnp.sort(v[i])
          return ray_body

        compute_A = make_ray_body(LbA, WbA, ObA)
        compute_B = make_ray_body(LbB, WbB, ObB)
        npair = nch // 2

        def in_slices(row0):
            return (len_hbm.at[pl.ds(row0 * P, CH * P)],
                    w_hbm.at[pl.ds(row0 * P, CH * P)])

        def out_slice(row0):
            return out_hbm.at[pl.ds(row0, CH)]

        ls0, ws0 = in_slices(base)
        pltpu.async_copy(ls0, LbA, isemA)
        pltpu.async_copy(ws0, WbA, isemA)

        def pair_body(j, carry):
            row_a = base + (2 * j) * CH
            row_b = row_a + CH
            lsb, wsb = in_slices(row_b)
            pltpu.async_copy(lsb, LbB, isemB)
            pltpu.async_copy(wsb, WbB, isemB)
            lsa, wsa = in_slices(row_a)
            pltpu.make_async_copy(lsa, LbA, isemA).wait()
            pltpu.make_async_copy(wsa, WbA, isemA).wait()
            @pl.when(j > 0)
            def _():
                pltpu.make_async_copy(ObA, out_slice(row_a - 2 * CH),
                                      osemA).wait()
            plsc.parallel_loop(0, CH, unroll=3)(compute_A)
            pltpu.async_copy(ObA, out_slice(row_a), osemA)
            @pl.when(j < npair - 1)
            def _():
                lsn, wsn = in_slices(row_a + 2 * CH)
                pltpu.async_copy(lsn, LbA, isemA)
                pltpu.async_copy(wsn, WbA, isemA)
            pltpu.make_async_copy(lsb, LbB, isemB).wait()
            pltpu.make_async_copy(wsb, WbB, isemB).wait()
            @pl.when(j > 0)
            def _():
                pltpu.make_async_copy(ObB, out_slice(row_b - 2 * CH),
                                      osemB).wait()
            plsc.parallel_loop(0, CH, unroll=3)(compute_B)
            pltpu.async_copy(ObB, out_slice(row_b), osemB)
            return carry

        lax.fori_loop(0, npair, pair_body, 0)
        last_a = base + (nch - 2) * CH
        pltpu.make_async_copy(ObA, out_slice(last_a), osemA).wait()
        pltpu.make_async_copy(ObB, out_slice(last_a + CH), osemB).wait()

    f = pl.kernel(
        body,
        out_type=jax.ShapeDtypeStruct((R, NS_OUT), jnp.float32),
        mesh=mesh,
        compiler_params=pltpu.CompilerParams(needs_layout_passes=False),
        scratch_types=[
            pltpu.VMEM((CH * P,), jnp.float32),
            pltpu.VMEM((CH * P,), jnp.float32),
            pltpu.VMEM((CH, NS_OUT), jnp.float32),
            pltpu.VMEM((CH * P,), jnp.float32),
            pltpu.VMEM((CH * P,), jnp.float32),
            pltpu.VMEM((CH, NS_OUT), jnp.float32),
            pltpu.VMEM((CH * P,), jnp.float32),
            pltpu.VMEM((CH * P,), jnp.float32),
            pltpu.VMEM((CH * P,), jnp.int32),
            pltpu.SemaphoreType.DMA,
            pltpu.SemaphoreType.DMA,
            pltpu.SemaphoreType.DMA,
            pltpu.SemaphoreType.DMA,
        ],
    )
    return f(Lf, Wf)


def kernel(origins, directions, lengths, ray_weights, xys):
    B, R, _ = lengths.shape
    z = _sampler_call(lengths.reshape(R * P), ray_weights.reshape(R * P))
    return (origins, directions, z.reshape(B, R, NS_OUT), xys)


# speedup vs baseline: 1.3997x; 1.3997x over previous
import jax
import jax.numpy as jnp
from jax import lax
from jax.experimental import pallas as pl
from jax.experimental.pallas import tpu as pltpu
from jax.experimental.pallas import tpu_sc as plsc

EPS = 1e-5
P = 64
NS_OUT = 128
NC, NSUB = 2, 16
NW = NC * NSUB
CH = 128


def _sampler_call(Lf, Wf):
    R = Lf.shape[0] // P
    rpw = R // NW
    nch = rpw // CH

    mesh = plsc.VectorSubcoreMesh(
        core_axis_name="c", subcore_axis_name="s",
        num_cores=NC, num_subcores=NSUB)

    def body(len_hbm, w_hbm, out_hbm, LbA, WbA, ObA, LbB, WbB, ObB,
             cdfb, midsb, histb, isemA, isemB, osemA, osemB):
        wid = lax.axis_index("s") * NC + lax.axis_index("c")
        base = wid * rpw

        iota = lax.iota(jnp.int32, 16)
        iof = iota.astype(jnp.float32)
        u = [iof * (1.0 / 63.0) + (16.0 * c / 63.0) for c in range(4)]
        ones_i = jnp.ones((16,), jnp.int32)
        zeros_i = jnp.zeros((16,), jnp.int32)
        ones_f = jnp.ones((16,), jnp.float32)
        full62 = jnp.full((16,), 62, jnp.int32)
        full63 = jnp.full((16,), 63, jnp.int32)
        mask3 = iota < 15

        def make_ray_body(Lb, Wb, Ob):
          def ray_body(r):
            rb = r * P
            rbv = jnp.full((16,), rb, jnp.int32)
            l = [Lb[pl.ds(rb + 16 * c, 16)] for c in range(4)]
            lsh = [Lb[pl.ds(rb + 16 * c + 1, 16)] for c in range(3)]
            lsh.append(plsc.load_gather(
                Lb, [rbv + jnp.minimum(iota + 49, 63)]))
            for c in range(4):
                midsb[pl.ds(rb + 16 * c, 16)] = 0.5 * (l[c] + lsh[c])
            cs = []
            for c in range(4):
                w = Wb[pl.ds(rb + 16 * c, 16)] + EPS
                cc = plsc.cumsum(w)
                cdfb[pl.ds(rb + 16 * c, 16)] = cc
                cs.append(cc)
            s0 = plsc.load_gather(cdfb, [rbv])
            g15 = plsc.load_gather(cdfb, [rbv + 15])
            g31 = plsc.load_gather(cdfb, [rbv + 31])
            g47 = plsc.load_gather(cdfb, [rbv + 47])
            g62 = plsc.load_gather(cdfb, [rbv + 62])
            off = [None, g15, g15 + g31, g15 + g31 + g47]
            invv = ones_f / ((off[3] + g62) - s0)
            pos = []
            for c in range(4):
                if c == 0:
                    cdf_c = (cs[c] - s0) * invv
                else:
                    cdf_c = (cs[c] + (off[c] - s0)) * invv
                cdfb[pl.ds(rb + 16 * c, 16)] = cdf_c
                p = cdf_c * 63.0
                ti = p.astype(jnp.int32)
                tf = ti.astype(jnp.float32)
                po = ti + jnp.where(tf < p, ones_i, zeros_i)
                po = jnp.minimum(po, full63)
                pos.append(po)
            for c in range(4):
                histb[pl.ds(rb + 16 * c, 16)] = zeros_i
            for c in range(4):
                plsc.addupdate_scatter(
                    histb, [rbv + pos[c]], ones_i,
                    mask=None if c < 3 else mask3)
            ics = []
            for c in range(4):
                ic = plsc.cumsum(histb[pl.ds(rb + 16 * c, 16)])
                histb[pl.ds(rb + 16 * c, 16)] = ic
                ics.append(ic)
            h15 = plsc.load_gather(histb, [rbv + 15])
            h31 = plsc.load_gather(histb, [rbv + 31])
            h47 = plsc.load_gather(histb, [rbv + 47])
            ioff = [None, h15, h15 + h31, h15 + h31 + h47]
            samples = []
            for c in range(4):
                ind = ics[c] if c == 0 else ics[c] + ioff[c]
                below = rbv + (ind - ones_i)
                above = rbv + jnp.minimum(ind, full62)
                g0 = plsc.load_gather(cdfb, [below])
                g1 = plsc.load_gather(cdfb, [above])
                m0 = plsc.load_gather(midsb, [below])
                m1 = plsc.load_gather(midsb, [above])
                den = g1 - g0
                den = jnp.where(den < EPS, ones_f, den)
                t = (u[c] - g0) / den
                samples.append(m0 + t * (m1 - m0))
            v = l + [lax.rev(samples[3], (0,)), lax.rev(samples[2], (0,)),
                     lax.rev(samples[1], (0,)), lax.rev(samples[0], (0,))]
            for i in range(4):
                a, b = v[i], v[i + 4]
                v[i], v[i + 4] = jnp.minimum(a, b), jnp.maximum(a, b)
            for h in (0, 4):
                for i in (0, 1):
                    a, b = v[h + i], v[h + i + 2]
                    v[h + i], v[h + i + 2] = jnp.minimum(a, b), jnp.maximum(a, b)
            for i in (0, 2, 4, 6):
                a, b = v[i], v[i + 1]
                v[i], v[i + 1] = jnp.minimum(a, b), jnp.maximum(a, b)
            for i in range(8):
                Ob[r, pl.ds(16 * i, 16)] = jnp.sort(v[i])
          return ray_body

        compute_A = make_ray_body(LbA, WbA, ObA)
        compute_B = make_ray_body(LbB, WbB, ObB)
        npair = nch // 2

        def in_slices(row0):
            return (len_hbm.at[pl.ds(row0 * P, CH * P)],
                    w_hbm.at[pl.ds(row0 * P, CH * P)])

        def out_slice(row0):
            return out_hbm.at[pl.ds(row0, CH)]

        ls0, ws0 = in_slices(base)
        pltpu.async_copy(ls0, LbA, isemA)
        pltpu.async_copy(ws0, WbA, isemA)

        def pair_body(j, carry):
            row_a = base + (2 * j) * CH
            row_b = row_a + CH
            lsb, wsb = in_slices(row_b)
            pltpu.async_copy(lsb, LbB, isemB)
            pltpu.async_copy(wsb, WbB, isemB)
            lsa, wsa = in_slices(row_a)
            pltpu.make_async_copy(lsa, LbA, isemA).wait()
            pltpu.make_async_copy(wsa, WbA, isemA).wait()
            @pl.when(j > 0)
            def _():
                pltpu.make_async_copy(ObA, out_slice(row_a - 2 * CH),
                                      osemA).wait()
            plsc.parallel_loop(0, CH, unroll=2)(compute_A)
            pltpu.async_copy(ObA, out_slice(row_a), osemA)
            @pl.when(j < npair - 1)
            def _():
                lsn, wsn = in_slices(row_a + 2 * CH)
                pltpu.async_copy(lsn, LbA, isemA)
                pltpu.async_copy(wsn, WbA, isemA)
            pltpu.make_async_copy(lsb, LbB, isemB).wait()
            pltpu.make_async_copy(wsb, WbB, isemB).wait()
            @pl.when(j > 0)
            def _():
                pltpu.make_async_copy(ObB, out_slice(row_b - 2 * CH),
                                      osemB).wait()
            plsc.parallel_loop(0, CH, unroll=2)(compute_B)
            pltpu.async_copy(ObB, out_slice(row_b), osemB)
            return carry

        lax.fori_loop(0, npair, pair_body, 0)
        last_a = base + (nch - 2) * CH
        pltpu.make_async_copy(ObA, out_slice(last_a), osemA).wait()
        pltpu.make_async_copy(ObB, out_slice(last_a + CH), osemB).wait()

    f = pl.kernel(
        body,
        out_type=jax.ShapeDtypeStruct((R, NS_OUT), jnp.float32),
        mesh=mesh,
        compiler_params=pltpu.CompilerParams(needs_layout_passes=False),
        scratch_types=[
            pltpu.VMEM((CH * P,), jnp.float32),
            pltpu.VMEM((CH * P,), jnp.float32),
            pltpu.VMEM((CH, NS_OUT), jnp.float32),
            pltpu.VMEM((CH * P,), jnp.float32),
            pltpu.VMEM((CH * P,), jnp.float32),
            pltpu.VMEM((CH, NS_OUT), jnp.float32),
            pltpu.VMEM((CH * P,), jnp.float32),
            pltpu.VMEM((CH * P,), jnp.float32),
            pltpu.VMEM((CH * P,), jnp.int32),
            pltpu.SemaphoreType.DMA,
            pltpu.SemaphoreType.DMA,
            pltpu.SemaphoreType.DMA,
            pltpu.SemaphoreType.DMA,
        ],
    )
    return f(Lf, Wf)


def kernel(origins, directions, lengths, ray_weights, xys):
    B, R, _ = lengths.shape
    z = _sampler_call(lengths.reshape(R * P), ray_weights.reshape(R * P))
    return (origins, directions, z.reshape(B, R, NS_OUT), xys)
